# Initial kernel scaffold; baseline (speedup 1.0000x reference)
import jax
import jax.numpy as jnp
from jax.experimental import pallas as pl


def kernel(x, edge_index, rbf, W1, b1, W2, b2):
    raise NotImplementedError("write your pallas kernel here")


# speedup vs baseline: 1.4392x; 1.4392x over previous
import functools

import jax
import jax.numpy as jnp
from jax import lax
from jax.experimental import pallas as pl
from jax.experimental.pallas import tpu as pltpu
from jax.experimental.pallas import tpu_sc as plsc

N = 50000
E = 800000
DIM = 64

NC = 2
NS = 16
LANES = 16

HALF = N // NC
ACC_ROWS = 25600
ZCHUNK = 80
OUT_PER_TILE = 1560

EDGES_PER_TILE = E // NS
C = 80
G = 80
NG = C // G
NCHUNK = EDGES_PER_TILE // C


def _mlp_block(rbf_ref, w1t_ref, b1_ref, w2t_ref, b2_ref, h_ref):
    r = rbf_ref[...]
    h1 = jnp.dot(r, w1t_ref[...], preferred_element_type=jnp.float32) + b1_ref[...]
    h1 = jnp.where(0.5 * h1 > 14.0, h1, 2.0 * jnp.log1p(jnp.exp(0.5 * h1)))
    h_ref[...] = jnp.dot(h1, w2t_ref[...], preferred_element_type=jnp.float32) + b2_ref[...]


def _edge_mlp(rbf, w1t, b1r, w2t, b2r):
    BE = 8000
    return pl.pallas_call(
        _mlp_block,
        grid=(E // BE,),
        in_specs=[
            pl.BlockSpec((BE, DIM), lambda i: (i, 0)),
            pl.BlockSpec((DIM, DIM), lambda i: (0, 0)),
            pl.BlockSpec((1, DIM), lambda i: (0, 0)),
            pl.BlockSpec((DIM, DIM), lambda i: (0, 0)),
            pl.BlockSpec((1, DIM), lambda i: (0, 0)),
        ],
        out_specs=pl.BlockSpec((BE, DIM), lambda i: (i, 0)),
        out_shape=jax.ShapeDtypeStruct((E, DIM), jnp.float32),
    )(rbf, w1t, b1r, w2t, b2r)


def _sc_body(x_hbm, src_hbm, dst_hbm, h_hbm, out_hbm,
             x_rows, h_rows, src_v, dst_v, idx2d, acc, sem):
    cid = lax.axis_index("c")
    sid = lax.axis_index("s")

    def zrow(r, carry):
        for q in range(DIM // LANES):
            x_rows[r, pl.ds(q * LANES, LANES)] = jnp.zeros((LANES,), jnp.float32)
        return carry
    lax.fori_loop(0, C, zrow, 0)
    zc_per_tile = ACC_ROWS // (NS * ZCHUNK)

    def zcopy(k, carry):
        pltpu.sync_copy(x_rows.at[pl.ds(0, ZCHUNK)],
                        acc.at[pl.ds(sid * (zc_per_tile * ZCHUNK) + k * ZCHUNK,
                                     ZCHUNK)])
        return carry
    lax.fori_loop(0, zc_per_tile, zcopy, 0)
    plsc.subcore_barrier()

    c_lo = cid * HALF

    def chunk(j, carry):
        base = sid * EDGES_PER_TILE + j * C
        pltpu.sync_copy(src_hbm.at[pl.ds(base, C)], src_v)
        pltpu.sync_copy(dst_hbm.at[pl.ds(base, C)], dst_v)
        pltpu.sync_copy(h_hbm.at[pl.ds(base, C)], h_rows)
        cps = [
            pltpu.async_copy(x_hbm.at[src_v.at[pl.ds(G * g, G)]],
                             x_rows.at[pl.ds(G * g, G)], sem)
            for g in range(NG)
        ]
        for v in range(C // LANES):
            d = dst_v[pl.ds(LANES * v, LANES)]
            local = d - c_lo
            valid = (local >= 0) & (local < HALF)
            idx = jnp.where(valid, local, HALF)
            vpg = G // LANES
            idx2d[v // vpg, pl.ds((v % vpg) * LANES, LANES)] = idx
        for cp in cps:
            cp.wait()
        def mrow(r, inner):
            for q in range(DIM // LANES):
                sl = pl.ds(q * LANES, LANES)
                x_rows[r, sl] = x_rows[r, sl] * h_rows[r, sl]
            return inner
        lax.fori_loop(0, C, mrow, 0)
        for g in range(NG):
            pltpu.sync_copy(x_rows.at[pl.ds(G * g, G)], acc.at[idx2d.at[g]],
                            add=True)
        return carry
    lax.fori_loop(0, NCHUNK, chunk, 0)

    plsc.subcore_barrier()
    pltpu.sync_copy(acc.at[pl.ds(sid * OUT_PER_TILE, OUT_PER_TILE)],
                    out_hbm.at[pl.ds(c_lo + sid * OUT_PER_TILE, OUT_PER_TILE)])
    @pl.when(sid == 0)
    def _tail():
        pltpu.sync_copy(acc.at[pl.ds(NS * OUT_PER_TILE, HALF - NS * OUT_PER_TILE)],
                        out_hbm.at[pl.ds(c_lo + NS * OUT_PER_TILE,
                                         HALF - NS * OUT_PER_TILE)])


@functools.partial(
    pl.kernel,
    out_type=jax.ShapeDtypeStruct((N, DIM), jnp.float32),
    mesh=plsc.VectorSubcoreMesh(core_axis_name="c", subcore_axis_name="s"),
    scratch_types=[
        pltpu.VMEM((C, DIM), jnp.float32),
        pltpu.VMEM((C, DIM), jnp.float32),
        pltpu.VMEM((C,), jnp.int32),
        pltpu.VMEM((C,), jnp.int32),
        pltpu.VMEM((NG, G), jnp.int32),
        pltpu.VMEM_SHARED((ACC_ROWS, DIM), jnp.float32),
        pltpu.SemaphoreType.DMA,
    ],
    compiler_params=pltpu.CompilerParams(use_tc_tiling_on_sc=False),
)
def _sc_gather_mul_scatter(x_hbm, src_hbm, dst_hbm, h_hbm, out_hbm, *scratch):
    _sc_body(x_hbm, src_hbm, dst_hbm, h_hbm, out_hbm, *scratch)


def kernel(x, edge_index, rbf, W1, b1, W2, b2):
    h = _edge_mlp(rbf, W1.T, b1.reshape(1, DIM), W2.T, b2.reshape(1, DIM))
    src = edge_index[0].astype(jnp.int32)
    dst = edge_index[1].astype(jnp.int32)
    return _sc_gather_mul_scatter(x, src, dst, h)


# speedup vs baseline: 1.8622x; 1.2939x over previous
import functools

import jax
import jax.numpy as jnp
from jax import lax
from jax.experimental import pallas as pl
from jax.experimental.pallas import tpu as pltpu
from jax.experimental.pallas import tpu_sc as plsc

N = 50000
E = 800000
DIM = 64

NC = 2
NS = 16
LANES = 16

HDIM = DIM // NC

EDGES_PER_TILE = E // NS
C = 400
G = 80
NG = C // G
NCHUNK = EDGES_PER_TILE // C

OUT_PER_TILE = 3120
ZFULL = N // C // NS
ZTAIL = N // NS - ZFULL * C


def _mlp_block(rbf_ref, w1t_ref, b1_ref, w2t_ref, b2_ref, h0_ref, h1_ref):
    r = rbf_ref[...]
    h1 = jnp.dot(r, w1t_ref[...], preferred_element_type=jnp.float32) + b1_ref[...]
    h1 = jnp.where(0.5 * h1 > 14.0, h1, 2.0 * jnp.log1p(jnp.exp(0.5 * h1)))
    h = jnp.dot(h1, w2t_ref[...], preferred_element_type=jnp.float32) + b2_ref[...]
    h0_ref[...] = h[:, :HDIM]
    h1_ref[...] = h[:, HDIM:]


def _edge_mlp(rbf, w1t, b1r, w2t, b2r):
    BE = 8000
    return pl.pallas_call(
        _mlp_block,
        grid=(E // BE,),
        in_specs=[
            pl.BlockSpec((BE, DIM), lambda i: (i, 0)),
            pl.BlockSpec((DIM, DIM), lambda i: (0, 0)),
            pl.BlockSpec((1, DIM), lambda i: (0, 0)),
            pl.BlockSpec((DIM, DIM), lambda i: (0, 0)),
            pl.BlockSpec((1, DIM), lambda i: (0, 0)),
        ],
        out_specs=[
            pl.BlockSpec((BE, HDIM), lambda i: (i, 0)),
            pl.BlockSpec((BE, HDIM), lambda i: (i, 0)),
        ],
        out_shape=[
            jax.ShapeDtypeStruct((E, HDIM), jnp.float32),
            jax.ShapeDtypeStruct((E, HDIM), jnp.float32),
        ],
    )(rbf, w1t, b1r, w2t, b2r)


def _sc_half(x_hbm, src_hbm, dst_hbm, h_hbm, out_hbm,
             x_rows, h_rows, src_v, dst2d, acc, sem, sid):
    def zrow(r, carry):
        for q in range(HDIM // LANES):
            x_rows[r, pl.ds(q * LANES, LANES)] = jnp.zeros((LANES,), jnp.float32)
        return carry
    lax.fori_loop(0, C, zrow, 0)
    zbase = sid * (N // NS)

    def zcopy(k, carry):
        pltpu.sync_copy(x_rows.at[pl.ds(0, C)], acc.at[pl.ds(zbase + k * C, C)])
        return carry
    lax.fori_loop(0, ZFULL, zcopy, 0)
    pltpu.sync_copy(x_rows.at[pl.ds(0, ZTAIL)],
                    acc.at[pl.ds(zbase + ZFULL * C, ZTAIL)])
    plsc.subcore_barrier()

    def chunk(j, carry):
        base = sid * EDGES_PER_TILE + j * C
        pltpu.sync_copy(src_hbm.at[pl.ds(base, C)], src_v)
        for g in range(NG):
            pltpu.sync_copy(dst_hbm.at[pl.ds(base + G * g, G)], dst2d.at[g])
        cps = [
            pltpu.async_copy(x_hbm.at[src_v.at[pl.ds(G * g, G)]],
                             x_rows.at[pl.ds(G * g, G)], sem)
            for g in range(NG)
        ]
        pltpu.sync_copy(h_hbm.at[pl.ds(base, C)], h_rows)
        for cp in cps:
            cp.wait()

        def mrow(r, inner):
            for q in range(HDIM // LANES):
                sl = pl.ds(q * LANES, LANES)
                x_rows[r, sl] = x_rows[r, sl] * h_rows[r, sl]
            return inner
        lax.fori_loop(0, C, mrow, 0)
        for g in range(NG):
            pltpu.sync_copy(x_rows.at[pl.ds(G * g, G)], acc.at[dst2d.at[g]],
                            add=True)
        return carry
    lax.fori_loop(0, NCHUNK, chunk, 0)

    plsc.subcore_barrier()
    pltpu.sync_copy(acc.at[pl.ds(sid * OUT_PER_TILE, OUT_PER_TILE)],
                    out_hbm.at[pl.ds(sid * OUT_PER_TILE, OUT_PER_TILE)])
    @pl.when(sid == 0)
    def _tail():
        pltpu.sync_copy(acc.at[pl.ds(NS * OUT_PER_TILE, N - NS * OUT_PER_TILE)],
                        out_hbm.at[pl.ds(NS * OUT_PER_TILE, N - NS * OUT_PER_TILE)])


@functools.partial(
    pl.kernel,
    out_type=[
        jax.ShapeDtypeStruct((N, HDIM), jnp.float32),
        jax.ShapeDtypeStruct((N, HDIM), jnp.float32),
    ],
    mesh=plsc.VectorSubcoreMesh(core_axis_name="c", subcore_axis_name="s"),
    scratch_types=[
        pltpu.VMEM((C, HDIM), jnp.float32),
        pltpu.VMEM((C, HDIM), jnp.float32),
        pltpu.VMEM((C,), jnp.int32),
        pltpu.VMEM((NG, G), jnp.int32),
        pltpu.VMEM_SHARED((N, HDIM), jnp.float32),
        pltpu.SemaphoreType.DMA,
    ],
    compiler_params=pltpu.CompilerParams(use_tc_tiling_on_sc=False),
)
def _sc_gather_mul_scatter(x0_hbm, x1_hbm, src_hbm, dst_hbm, h0_hbm, h1_hbm,
                           out0_hbm, out1_hbm,
                           x_rows, h_rows, src_v, dst2d, acc, sem):
    cid = lax.axis_index("c")
    sid = lax.axis_index("s")

    @pl.when(cid == 0)
    def _half0():
        _sc_half(x0_hbm, src_hbm, dst_hbm, h0_hbm, out0_hbm,
                 x_rows, h_rows, src_v, dst2d, acc, sem, sid)

    @pl.when(cid == 1)
    def _half1():
        _sc_half(x1_hbm, src_hbm, dst_hbm, h1_hbm, out1_hbm,
                 x_rows, h_rows, src_v, dst2d, acc, sem, sid)


def kernel(x, edge_index, rbf, W1, b1, W2, b2):
    h0, h1 = _edge_mlp(rbf, W1.T, b1.reshape(1, DIM), W2.T, b2.reshape(1, DIM))
    src = edge_index[0].astype(jnp.int32)
    dst = edge_index[1].astype(jnp.int32)
    x0 = x[:, :HDIM]
    x1 = x[:, HDIM:]
    out0, out1 = _sc_gather_mul_scatter(x0, x1, src, dst, h0, h1)
    return jnp.concatenate([out0, out1], axis=1)


# speedup vs baseline: 2.0320x; 1.0912x over previous
import functools

import jax
import jax.numpy as jnp
from jax import lax
from jax.experimental import pallas as pl
from jax.experimental.pallas import tpu as pltpu
from jax.experimental.pallas import tpu_sc as plsc

N = 50000
E = 800000
DIM = 64

NC = 2
NS = 16
LANES = 16

HDIM = DIM // NC

BE = 8000
FG = BE // 4
NBLK = E // BE

CH = 320
RH = CH // 4
G = 80
CPB = BE // CH
TOTCH = E // CH

OUT_PER_TILE = 3120
ZROWS = 320
ZFULL = (N // NS) // ZROWS
ZTAIL = N // NS - ZFULL * ZROWS


def _mlp_block(rbf_ref, w1t_ref, b1_ref, w2t_ref, b2_ref, h0_ref, h1_ref):
    r = rbf_ref[...]
    h1 = jnp.dot(r, w1t_ref[...], preferred_element_type=jnp.float32) + b1_ref[...]
    h1 = jnp.where(0.5 * h1 > 14.0, h1, 2.0 * jnp.log1p(jnp.exp(0.5 * h1)))
    h = jnp.dot(h1, w2t_ref[...], preferred_element_type=jnp.float32) + b2_ref[...]
    h0_ref[...] = jnp.concatenate(
        [h[g * FG:(g + 1) * FG, :HDIM] for g in range(4)], axis=1)
    h1_ref[...] = jnp.concatenate(
        [h[g * FG:(g + 1) * FG, HDIM:] for g in range(4)], axis=1)


def _edge_mlp(rbf, w1t, b1r, w2t, b2r):
    return pl.pallas_call(
        _mlp_block,
        grid=(NBLK,),
        in_specs=[
            pl.BlockSpec((BE, DIM), lambda i: (i, 0)),
            pl.BlockSpec((DIM, DIM), lambda i: (0, 0)),
            pl.BlockSpec((1, DIM), lambda i: (0, 0)),
            pl.BlockSpec((DIM, DIM), lambda i: (0, 0)),
            pl.BlockSpec((1, DIM), lambda i: (0, 0)),
        ],
        out_specs=[
            pl.BlockSpec((FG, 128), lambda i: (i, 0)),
            pl.BlockSpec((FG, 128), lambda i: (i, 0)),
        ],
        out_shape=[
            jax.ShapeDtypeStruct((NBLK * FG, 128), jnp.float32),
            jax.ShapeDtypeStruct((NBLK * FG, 128), jnp.float32),
        ],
    )(rbf, w1t, b1r, w2t, b2r)


def _sc_half(x_hbm, src_hbm, dst_hbm, h_hbm, out_hbm,
             x_rows, h_rows, src_v, dst2d, acc, sem, sid):
    def zrow(r, carry):
        for q in range(HDIM // LANES):
            x_rows[r, pl.ds(q * LANES, LANES)] = jnp.zeros((LANES,), jnp.float32)
        return carry
    lax.fori_loop(0, ZROWS, zrow, 0)
    zbase = sid * (N // NS)

    def zcopy(k, carry):
        pltpu.sync_copy(x_rows.at[pl.ds(0, ZROWS)],
                        acc.at[pl.ds(zbase + k * ZROWS, ZROWS)])
        return carry
    lax.fori_loop(0, ZFULL, zcopy, 0)
    pltpu.sync_copy(x_rows.at[pl.ds(0, ZTAIL)],
                    acc.at[pl.ds(zbase + ZFULL * ZROWS, ZTAIL)])
    plsc.subcore_barrier()

    nk = TOTCH // NS + jnp.where(sid < TOTCH % NS, 1, 0)

    def chunk(kk, carry):
        k = kk * NS + sid
        b = k // CPB
        r0 = (k % CPB) * RH
        eb = b * BE + r0
        for g in range(4):
            pltpu.sync_copy(src_hbm.at[pl.ds(eb + g * FG, G)],
                            src_v.at[pl.ds(g * G, G)])
            pltpu.sync_copy(dst_hbm.at[pl.ds(eb + g * FG, G)], dst2d.at[g])
        cps = [
            pltpu.async_copy(x_hbm.at[src_v.at[pl.ds(G * g, G)]],
                             x_rows.at[pl.ds(G * g, G)], sem)
            for g in range(4)
        ]
        pltpu.sync_copy(h_hbm.at[pl.ds(b * FG + r0, RH)], h_rows)
        for cp in cps:
            cp.wait()

        def mrow(rr, inner):
            for g in range(4):
                for q in range(HDIM // LANES):
                    sl = pl.ds(q * LANES, LANES)
                    hsl = pl.ds(g * HDIM + q * LANES, LANES)
                    x_rows[g * G + rr, sl] = x_rows[g * G + rr, sl] * h_rows[rr, hsl]
            return inner
        lax.fori_loop(0, RH, mrow, 0)
        for g in range(4):
            pltpu.sync_copy(x_rows.at[pl.ds(G * g, G)], acc.at[dst2d.at[g]],
                            add=True)
        return carry
    lax.fori_loop(0, nk, chunk, 0)

    plsc.subcore_barrier()
    pltpu.sync_copy(acc.at[pl.ds(sid * OUT_PER_TILE, OUT_PER_TILE)],
                    out_hbm.at[pl.ds(sid * OUT_PER_TILE, OUT_PER_TILE)])
    @pl.when(sid == 0)
    def _tail():
        pltpu.sync_copy(acc.at[pl.ds(NS * OUT_PER_TILE, N - NS * OUT_PER_TILE)],
                        out_hbm.at[pl.ds(NS * OUT_PER_TILE, N - NS * OUT_PER_TILE)])


@functools.partial(
    pl.kernel,
    out_type=[
        jax.ShapeDtypeStruct((N, HDIM), jnp.float32),
        jax.ShapeDtypeStruct((N, HDIM), jnp.float32),
    ],
    mesh=plsc.VectorSubcoreMesh(core_axis_name="c", subcore_axis_name="s"),
    scratch_types=[
        pltpu.VMEM((CH, HDIM), jnp.float32),
        pltpu.VMEM((RH, 128), jnp.float32),
        pltpu.VMEM((CH,), jnp.int32),
        pltpu.VMEM((4, G), jnp.int32),
        pltpu.VMEM_SHARED((N, HDIM), jnp.float32),
        pltpu.SemaphoreType.DMA,
    ],
    compiler_params=pltpu.CompilerParams(use_tc_tiling_on_sc=False),
)
def _sc_gather_mul_scatter(x0_hbm, x1_hbm, src_hbm, dst_hbm, h0_hbm, h1_hbm,
                           out0_hbm, out1_hbm,
                           x_rows, h_rows, src_v, dst2d, acc, sem):
    cid = lax.axis_index("c")
    sid = lax.axis_index("s")

    @pl.when(cid == 0)
    def _half0():
        _sc_half(x0_hbm, src_hbm, dst_hbm, h0_hbm, out0_hbm,
                 x_rows, h_rows, src_v, dst2d, acc, sem, sid)

    @pl.when(cid == 1)
    def _half1():
        _sc_half(x1_hbm, src_hbm, dst_hbm, h1_hbm, out1_hbm,
                 x_rows, h_rows, src_v, dst2d, acc, sem, sid)


def kernel(x, edge_index, rbf, W1, b1, W2, b2):
    h0, h1 = _edge_mlp(rbf, W1.T, b1.reshape(1, DIM), W2.T, b2.reshape(1, DIM))
    src = edge_index[0].astype(jnp.int32)
    dst = edge_index[1].astype(jnp.int32)
    x0 = x[:, :HDIM]
    x1 = x[:, HDIM:]
    out0, out1 = _sc_gather_mul_scatter(x0, x1, src, dst, h0, h1)
    return jnp.concatenate([out0, out1], axis=1)


# speedup vs baseline: 4.0810x; 2.0083x over previous
import functools

import jax
import jax.numpy as jnp
from jax import lax
from jax.experimental import pallas as pl
from jax.experimental.pallas import tpu as pltpu
from jax.experimental.pallas import tpu_sc as plsc

N = 50000
E = 800000
DIM = 64

NC = 2
NS = 16
LANES = 16

HDIM = DIM // NC

BEB = 8192
GBLK = 25
FGG = BEB * GBLK
LASTBLK = (E - 1) // BEB

CH = 320
CPG = FGG // CH
TOTCH = E // CH

OUT_PER_TILE = 3120
ZROWS = 320
ZFULL = (N // NS) // ZROWS
ZTAIL = N // NS - ZFULL * ZROWS


def _mlp_block(r0_ref, r1_ref, r2_ref, r3_ref, w1_ref, b1_ref, w2_ref, b2_ref,
               h0_ref, h1_ref):
    halves0, halves1 = [], []
    for rt_ref in (r0_ref, r1_ref, r2_ref, r3_ref):
        r_t = rt_ref[...]
        a = jnp.dot(w1_ref[...], r_t, preferred_element_type=jnp.float32) + b1_ref[...]
        a = jnp.where(0.5 * a > 14.0, a, 2.0 * jnp.log1p(jnp.exp(0.5 * a)))
        h = lax.dot_general(a, w2_ref[...], (((0,), (1,)), ((), ())),
                            preferred_element_type=jnp.float32) + b2_ref[...]
        halves0.append(h[:, :HDIM])
        halves1.append(h[:, HDIM:])
    h0_ref[...] = jnp.concatenate(halves0, axis=1)
    h1_ref[...] = jnp.concatenate(halves1, axis=1)


def _edge_mlp(rbf_t, w1, b1c, w2, b2r):
    def _rspec(g):
        return pl.BlockSpec(
            (DIM, BEB), lambda i, g=g: (0, jnp.minimum(g * GBLK + i, LASTBLK)))
    return pl.pallas_call(
        _mlp_block,
        grid=(GBLK,),
        in_specs=[
            _rspec(0), _rspec(1), _rspec(2), _rspec(3),
            pl.BlockSpec((DIM, DIM), lambda i: (0, 0)),
            pl.BlockSpec((DIM, 1), lambda i: (0, 0)),
            pl.BlockSpec((DIM, DIM), lambda i: (0, 0)),
            pl.BlockSpec((1, DIM), lambda i: (0, 0)),
        ],
        out_specs=[
            pl.BlockSpec((BEB, 128), lambda i: (i, 0)),
            pl.BlockSpec((BEB, 128), lambda i: (i, 0)),
        ],
        out_shape=[
            jax.ShapeDtypeStruct((FGG, 128), jnp.float32),
            jax.ShapeDtypeStruct((FGG, 128), jnp.float32),
        ],
    )(rbf_t, rbf_t, rbf_t, rbf_t, w1, b1c, w2, b2r)


def _sc_half(x_hbm, src_hbm, dst_hbm, h_hbm, out_hbm,
             x_rows, h_rows, src_v, d0, d1, d2, acc, sem, sid):
    def zrow(r, carry):
        for q in range(HDIM // LANES):
            x_rows[r, pl.ds(q * LANES, LANES)] = jnp.zeros((LANES,), jnp.float32)
        return carry
    lax.fori_loop(0, ZROWS, zrow, 0)
    zbase = sid * (N // NS)

    def zcopy(k, carry):
        pltpu.sync_copy(x_rows.at[pl.ds(0, ZROWS)],
                        acc.at[pl.ds(zbase + k * ZROWS, ZROWS)])
        return carry
    lax.fori_loop(0, ZFULL, zcopy, 0)
    pltpu.sync_copy(x_rows.at[pl.ds(0, ZTAIL)],
                    acc.at[pl.ds(zbase + ZFULL * ZROWS, ZTAIL)])
    plsc.subcore_barrier()

    nk = TOTCH // NS + jnp.where(sid < TOTCH % NS, 1, 0)

    def chunk(kk, carry):
        k = kk * NS + sid
        gl = k // CPG
        j = k - gl * CPG
        base_e = gl * FGG + j * CH
        base_r = j * CH

        cps = [
            pltpu.async_copy(src_hbm.at[pl.ds(base_e, CH)], src_v, sem),
            pltpu.async_copy(dst_hbm.at[pl.ds(base_e, 128)], d0, sem),
            pltpu.async_copy(dst_hbm.at[pl.ds(base_e + 128, 128)], d1, sem),
            pltpu.async_copy(dst_hbm.at[pl.ds(base_e + 256, 64)], d2, sem),
            pltpu.async_copy(
                h_hbm.at[pl.ds(base_r, CH), pl.ds(gl * HDIM, HDIM)], h_rows, sem),
        ]
        for cp in cps:
            cp.wait()

        cps = [
            pltpu.async_copy(x_hbm.at[src_v.at[pl.ds(0, 128)]],
                             x_rows.at[pl.ds(0, 128)], sem),
            pltpu.async_copy(x_hbm.at[src_v.at[pl.ds(128, 128)]],
                             x_rows.at[pl.ds(128, 128)], sem),
            pltpu.async_copy(x_hbm.at[src_v.at[pl.ds(256, 64)]],
                             x_rows.at[pl.ds(256, 64)], sem),
        ]
        for cp in cps:
            cp.wait()

        def mrow(i, inner):
            for u in range(4):
                for q in range(HDIM // LANES):
                    sl = pl.ds(q * LANES, LANES)
                    x_rows[4 * i + u, sl] = x_rows[4 * i + u, sl] * h_rows[4 * i + u, sl]
            return inner
        lax.fori_loop(0, CH // 4, mrow, 0)

        cps = [
            pltpu.async_copy(x_rows.at[pl.ds(0, 128)], acc.at[d0], sem, add=True),
            pltpu.async_copy(x_rows.at[pl.ds(128, 128)], acc.at[d1], sem, add=True),
            pltpu.async_copy(x_rows.at[pl.ds(256, 64)], acc.at[d2], sem, add=True),
        ]
        for cp in cps:
            cp.wait()
        return carry
    lax.fori_loop(0, nk, chunk, 0)

    plsc.subcore_barrier()
    pltpu.sync_copy(acc.at[pl.ds(sid * OUT_PER_TILE, OUT_PER_TILE)],
                    out_hbm.at[pl.ds(sid * OUT_PER_TILE, OUT_PER_TILE)])
    @pl.when(sid == 0)
    def _tail():
        pltpu.sync_copy(acc.at[pl.ds(NS * OUT_PER_TILE, N - NS * OUT_PER_TILE)],
                        out_hbm.at[pl.ds(NS * OUT_PER_TILE, N - NS * OUT_PER_TILE)])


@functools.partial(
    pl.kernel,
    out_type=[
        jax.ShapeDtypeStruct((N, HDIM), jnp.float32),
        jax.ShapeDtypeStruct((N, HDIM), jnp.float32),
    ],
    mesh=plsc.VectorSubcoreMesh(core_axis_name="c", subcore_axis_name="s"),
    scratch_types=[
        pltpu.VMEM((CH, HDIM), jnp.float32),
        pltpu.VMEM((CH, HDIM), jnp.float32),
        pltpu.VMEM((CH,), jnp.int32),
        pltpu.VMEM((128,), jnp.int32),
        pltpu.VMEM((128,), jnp.int32),
        pltpu.VMEM((64,), jnp.int32),
        pltpu.VMEM_SHARED((N, HDIM), jnp.float32),
        pltpu.SemaphoreType.DMA,
    ],
    compiler_params=pltpu.CompilerParams(use_tc_tiling_on_sc=False),
)
def _sc_gather_mul_scatter(x0_hbm, x1_hbm, src_hbm, dst_hbm, h0_hbm, h1_hbm,
                           out0_hbm, out1_hbm,
                           x_rows, h_rows, src_v, d0, d1, d2, acc, sem):
    cid = lax.axis_index("c")
    sid = lax.axis_index("s")

    @pl.when(cid == 0)
    def _half0():
        _sc_half(x0_hbm, src_hbm, dst_hbm, h0_hbm, out0_hbm,
                 x_rows, h_rows, src_v, d0, d1, d2, acc, sem, sid)

    @pl.when(cid == 1)
    def _half1():
        _sc_half(x1_hbm, src_hbm, dst_hbm, h1_hbm, out1_hbm,
                 x_rows, h_rows, src_v, d0, d1, d2, acc, sem, sid)


def kernel(x, edge_index, rbf, W1, b1, W2, b2):
    h0, h1 = _edge_mlp(rbf.T, W1, b1.reshape(DIM, 1), W2, b2.reshape(1, DIM))
    src = edge_index[0].astype(jnp.int32)
    dst = edge_index[1].astype(jnp.int32)
    x0 = x[:, :HDIM]
    x1 = x[:, HDIM:]
    out0, out1 = _sc_gather_mul_scatter(x0, x1, src, dst, h0, h1)
    return jnp.concatenate([out0, out1], axis=1)


# speedup vs baseline: 4.0915x; 1.0026x over previous
import functools

import jax
import jax.numpy as jnp
from jax import lax
from jax.experimental import pallas as pl
from jax.experimental.pallas import tpu as pltpu
from jax.experimental.pallas import tpu_sc as plsc

N = 50000
E = 800000
DIM = 64

NC = 2
NS = 16
LANES = 16

HDIM = DIM // NC

BEB = 8192
GBLK = 15
FGG = BEB * GBLK
LASTBLK = (E - 1) // BEB

PHASE_BASE_BLK = (0, 50)
PHASE_BASE_E = (0, 50 * BEB)
PHASE_EDGES = (50 * BEB, E - 50 * BEB)

CH = 320
CPG = FGG // CH

OUT_PER_TILE = 3120
ZROWS = 320
ZFULL = (N // NS) // ZROWS
ZTAIL = N // NS - ZFULL * ZROWS


def _mlp_block(r0_ref, r1_ref, r2_ref, r3_ref, w1_ref, b1_ref, w2_ref, b2_ref,
               h0_ref, h1_ref):
    halves0, halves1 = [], []
    for rt_ref in (r0_ref, r1_ref, r2_ref, r3_ref):
        r_t = rt_ref[...]
        a = jnp.dot(w1_ref[...], r_t, preferred_element_type=jnp.float32) + b1_ref[...]
        a = jnp.where(0.5 * a > 14.0, a, 2.0 * jnp.log1p(jnp.exp(0.5 * a)))
        h = lax.dot_general(a, w2_ref[...], (((0,), (1,)), ((), ())),
                            preferred_element_type=jnp.float32) + b2_ref[...]
        halves0.append(h[:, :HDIM])
        halves1.append(h[:, HDIM:])
    h0_ref[...] = jnp.concatenate(halves0, axis=1)
    h1_ref[...] = jnp.concatenate(halves1, axis=1)


def _make_edge_mlp(base_blk):
    def _rspec(g):
        return pl.BlockSpec(
            (DIM, BEB),
            lambda i, g=g: (0, jnp.minimum(base_blk + g * GBLK + i, LASTBLK)))

    def run(rbf_t, w1, b1c, w2, b2r):
        return pl.pallas_call(
            _mlp_block,
            grid=(GBLK,),
            in_specs=[
                _rspec(0), _rspec(1), _rspec(2), _rspec(3),
                pl.BlockSpec((DIM, DIM), lambda i: (0, 0)),
                pl.BlockSpec((DIM, 1), lambda i: (0, 0)),
                pl.BlockSpec((DIM, DIM), lambda i: (0, 0)),
                pl.BlockSpec((1, DIM), lambda i: (0, 0)),
            ],
            out_specs=[
                pl.BlockSpec((BEB, 128), lambda i: (i, 0)),
                pl.BlockSpec((BEB, 128), lambda i: (i, 0)),
            ],
            out_shape=[
                jax.ShapeDtypeStruct((FGG, 128), jnp.float32),
                jax.ShapeDtypeStruct((FGG, 128), jnp.float32),
            ],
        )(rbf_t, rbf_t, rbf_t, rbf_t, w1, b1c, w2, b2r)
    return run


def _sc_half(x_hbm, src_hbm, dst_hbm, h_hbm, out_hbm,
             x_rows, h_rows, src_v, d0, d1, d2, acc, sem, sid,
             base_e, totch):
    def zrow(r, carry):
        for q in range(HDIM // LANES):
            x_rows[r, pl.ds(q * LANES, LANES)] = jnp.zeros((LANES,), jnp.float32)
        return carry
    lax.fori_loop(0, ZROWS, zrow, 0)
    zbase = sid * (N // NS)

    def zcopy(k, carry):
        pltpu.sync_copy(x_rows.at[pl.ds(0, ZROWS)],
                        acc.at[pl.ds(zbase + k * ZROWS, ZROWS)])
        return carry
    lax.fori_loop(0, ZFULL, zcopy, 0)
    pltpu.sync_copy(x_rows.at[pl.ds(0, ZTAIL)],
                    acc.at[pl.ds(zbase + ZFULL * ZROWS, ZTAIL)])
    plsc.subcore_barrier()

    nk = totch // NS + jnp.where(sid < totch % NS, 1, 0)

    def chunk(kk, carry):
        k = kk * NS + sid
        gl = k // CPG
        j = k - gl * CPG
        base = base_e + gl * FGG + j * CH
        base_r = j * CH

        cps = [
            pltpu.async_copy(src_hbm.at[pl.ds(base, CH)], src_v, sem),
            pltpu.async_copy(dst_hbm.at[pl.ds(base, 128)], d0, sem),
            pltpu.async_copy(dst_hbm.at[pl.ds(base + 128, 128)], d1, sem),
            pltpu.async_copy(dst_hbm.at[pl.ds(base + 256, 64)], d2, sem),
            pltpu.async_copy(
                h_hbm.at[pl.ds(base_r, CH), pl.ds(gl * HDIM, HDIM)], h_rows, sem),
        ]
        for cp in cps:
            cp.wait()

        cps = [
            pltpu.async_copy(x_hbm.at[src_v.at[pl.ds(0, 128)]],
                             x_rows.at[pl.ds(0, 128)], sem),
            pltpu.async_copy(x_hbm.at[src_v.at[pl.ds(128, 128)]],
                             x_rows.at[pl.ds(128, 128)], sem),
            pltpu.async_copy(x_hbm.at[src_v.at[pl.ds(256, 64)]],
                             x_rows.at[pl.ds(256, 64)], sem),
        ]
        for cp in cps:
            cp.wait()

        def mrow(i, inner):
            for u in range(4):
                for q in range(HDIM // LANES):
                    sl = pl.ds(q * LANES, LANES)
                    x_rows[4 * i + u, sl] = x_rows[4 * i + u, sl] * h_rows[4 * i + u, sl]
            return inner
        lax.fori_loop(0, CH // 4, mrow, 0)

        cps = [
            pltpu.async_copy(x_rows.at[pl.ds(0, 128)], acc.at[d0], sem, add=True),
            pltpu.async_copy(x_rows.at[pl.ds(128, 128)], acc.at[d1], sem, add=True),
            pltpu.async_copy(x_rows.at[pl.ds(256, 64)], acc.at[d2], sem, add=True),
        ]
        for cp in cps:
            cp.wait()
        return carry
    lax.fori_loop(0, nk, chunk, 0)

    plsc.subcore_barrier()
    pltpu.sync_copy(acc.at[pl.ds(sid * OUT_PER_TILE, OUT_PER_TILE)],
                    out_hbm.at[pl.ds(sid * OUT_PER_TILE, OUT_PER_TILE)])
    @pl.when(sid == 0)
    def _tail():
        pltpu.sync_copy(acc.at[pl.ds(NS * OUT_PER_TILE, N - NS * OUT_PER_TILE)],
                        out_hbm.at[pl.ds(NS * OUT_PER_TILE, N - NS * OUT_PER_TILE)])


def _make_sc(phase):
    base_e = PHASE_BASE_E[phase]
    totch = PHASE_EDGES[phase] // CH

    @functools.partial(
        pl.kernel,
        out_type=[
            jax.ShapeDtypeStruct((N, HDIM), jnp.float32),
            jax.ShapeDtypeStruct((N, HDIM), jnp.float32),
        ],
        mesh=plsc.VectorSubcoreMesh(core_axis_name="c", subcore_axis_name="s"),
        scratch_types=[
            pltpu.VMEM((CH, HDIM), jnp.float32),
            pltpu.VMEM((CH, HDIM), jnp.float32),
            pltpu.VMEM((CH,), jnp.int32),
            pltpu.VMEM((128,), jnp.int32),
            pltpu.VMEM((128,), jnp.int32),
            pltpu.VMEM((64,), jnp.int32),
            pltpu.VMEM_SHARED((N, HDIM), jnp.float32),
            pltpu.SemaphoreType.DMA,
        ],
        compiler_params=pltpu.CompilerParams(use_tc_tiling_on_sc=False),
    )
    def sc(x0_hbm, x1_hbm, src_hbm, dst_hbm, h0_hbm, h1_hbm,
           out0_hbm, out1_hbm,
           x_rows, h_rows, src_v, d0, d1, d2, acc, sem):
        cid = lax.axis_index("c")
        sid = lax.axis_index("s")

        @pl.when(cid == 0)
        def _half0():
            _sc_half(x0_hbm, src_hbm, dst_hbm, h0_hbm, out0_hbm,
                     x_rows, h_rows, src_v, d0, d1, d2, acc, sem, sid,
                     base_e, totch)

        @pl.when(cid == 1)
        def _half1():
            _sc_half(x1_hbm, src_hbm, dst_hbm, h1_hbm, out1_hbm,
                     x_rows, h_rows, src_v, d0, d1, d2, acc, sem, sid,
                     base_e, totch)
    return sc


_mlp_a = _make_edge_mlp(PHASE_BASE_BLK[0])
_mlp_b = _make_edge_mlp(PHASE_BASE_BLK[1])
_sc_a = _make_sc(0)
_sc_b = _make_sc(1)


def kernel(x, edge_index, rbf, W1, b1, W2, b2):
    rbf_t = rbf.T
    b1c = b1.reshape(DIM, 1)
    b2r = b2.reshape(1, DIM)
    src = edge_index[0].astype(jnp.int32)
    dst = edge_index[1].astype(jnp.int32)
    x0 = x[:, :HDIM]
    x1 = x[:, HDIM:]
    ha0, ha1 = _mlp_a(rbf_t, W1, b1c, W2, b2r)
    hb0, hb1 = _mlp_b(rbf_t, W1, b1c, W2, b2r)
    oa0, oa1 = _sc_a(x0, x1, src, dst, ha0, ha1)
    ob0, ob1 = _sc_b(x0, x1, src, dst, hb0, hb1)
    return jnp.concatenate([oa0 + ob0, oa1 + ob1], axis=1)


# speedup vs baseline: 4.3841x; 1.0715x over previous
import functools

import jax
import jax.numpy as jnp
from jax import lax
from jax.experimental import pallas as pl
from jax.experimental.pallas import tpu as pltpu
from jax.experimental.pallas import tpu_sc as plsc

N = 50000
E = 800000
DIM = 64

NC = 2
NS = 16
LANES = 16

HDIM = DIM // NC

BEB = 8192
GBLK = 15
FGG = BEB * GBLK
LASTBLK = (E - 1) // BEB

PHASE_BASE_BLK = (0, 50)
PHASE_BASE_E = (0, 50 * BEB)
PHASE_EDGES = (50 * BEB, E - 50 * BEB)

CH = 320
CPG = FGG // CH

OUT_PER_TILE = 3120
ZROWS = 320
ZFULL = (N // NS) // ZROWS
ZTAIL = N // NS - ZFULL * ZROWS


def _mlp_block(r0_ref, r1_ref, r2_ref, r3_ref, w1_ref, b1_ref, w2_ref, b2_ref,
               h0_ref, h1_ref):
    halves0, halves1 = [], []
    for rt_ref in (r0_ref, r1_ref, r2_ref, r3_ref):
        r_t = rt_ref[...]
        a = jnp.dot(w1_ref[...], r_t, preferred_element_type=jnp.float32) + b1_ref[...]
        a = jnp.where(0.5 * a > 14.0, a, 2.0 * jnp.log1p(jnp.exp(0.5 * a)))
        h = lax.dot_general(a, w2_ref[...], (((0,), (1,)), ((), ())),
                            preferred_element_type=jnp.float32) + b2_ref[...]
        halves0.append(h[:, :HDIM])
        halves1.append(h[:, HDIM:])
    h0_ref[...] = jnp.concatenate(halves0, axis=1)
    h1_ref[...] = jnp.concatenate(halves1, axis=1)


def _make_edge_mlp(base_blk):
    def _rspec(g):
        return pl.BlockSpec(
            (DIM, BEB),
            lambda i, g=g: (0, jnp.minimum(base_blk + g * GBLK + i, LASTBLK)))

    def run(rbf_t, w1, b1c, w2, b2r):
        return pl.pallas_call(
            _mlp_block,
            grid=(GBLK,),
            in_specs=[
                _rspec(0), _rspec(1), _rspec(2), _rspec(3),
                pl.BlockSpec((DIM, DIM), lambda i: (0, 0)),
                pl.BlockSpec((DIM, 1), lambda i: (0, 0)),
                pl.BlockSpec((DIM, DIM), lambda i: (0, 0)),
                pl.BlockSpec((1, DIM), lambda i: (0, 0)),
            ],
            out_specs=[
                pl.BlockSpec((BEB, 128), lambda i: (i, 0)),
                pl.BlockSpec((BEB, 128), lambda i: (i, 0)),
            ],
            out_shape=[
                jax.ShapeDtypeStruct((FGG, 128), jnp.float32),
                jax.ShapeDtypeStruct((FGG, 128), jnp.float32),
            ],
        )(rbf_t, rbf_t, rbf_t, rbf_t, w1, b1c, w2, b2r)
    return run


def _sc_half(x_hbm, src_hbm, dst_hbm, h_hbm, out_hbm,
             x_rows, h_rows, src_v, d0, d1, d2, acc, sem, sid,
             base_e, totch, prev_hbm=None):
    zbase = sid * (N // NS)
    if prev_hbm is None:
        def zrow(r, carry):
            for q in range(HDIM // LANES):
                x_rows[r, pl.ds(q * LANES, LANES)] = jnp.zeros((LANES,), jnp.float32)
            return carry
        lax.fori_loop(0, ZROWS, zrow, 0)

        def zcopy(k, carry):
            pltpu.sync_copy(x_rows.at[pl.ds(0, ZROWS)],
                            acc.at[pl.ds(zbase + k * ZROWS, ZROWS)])
            return carry
        lax.fori_loop(0, ZFULL, zcopy, 0)
        pltpu.sync_copy(x_rows.at[pl.ds(0, ZTAIL)],
                        acc.at[pl.ds(zbase + ZFULL * ZROWS, ZTAIL)])
    else:
        pltpu.sync_copy(prev_hbm.at[pl.ds(zbase, N // NS)],
                        acc.at[pl.ds(zbase, N // NS)])
    plsc.subcore_barrier()

    nk = totch // NS + jnp.where(sid < totch % NS, 1, 0)

    def chunk(kk, carry):
        k = kk * NS + sid
        gl = k // CPG
        j = k - gl * CPG
        base = base_e + gl * FGG + j * CH
        base_r = j * CH

        cps = [
            pltpu.async_copy(src_hbm.at[pl.ds(base, CH)], src_v, sem),
            pltpu.async_copy(dst_hbm.at[pl.ds(base, 128)], d0, sem),
            pltpu.async_copy(dst_hbm.at[pl.ds(base + 128, 128)], d1, sem),
            pltpu.async_copy(dst_hbm.at[pl.ds(base + 256, 64)], d2, sem),
            pltpu.async_copy(
                h_hbm.at[pl.ds(base_r, CH), pl.ds(gl * HDIM, HDIM)], h_rows, sem),
        ]
        for cp in cps:
            cp.wait()

        cps = [
            pltpu.async_copy(x_hbm.at[src_v.at[pl.ds(0, 128)]],
                             x_rows.at[pl.ds(0, 128)], sem),
            pltpu.async_copy(x_hbm.at[src_v.at[pl.ds(128, 128)]],
                             x_rows.at[pl.ds(128, 128)], sem),
            pltpu.async_copy(x_hbm.at[src_v.at[pl.ds(256, 64)]],
                             x_rows.at[pl.ds(256, 64)], sem),
        ]
        for cp in cps:
            cp.wait()

        def mrow(i, inner):
            for u in range(4):
                for q in range(HDIM // LANES):
                    sl = pl.ds(q * LANES, LANES)
                    x_rows[4 * i + u, sl] = x_rows[4 * i + u, sl] * h_rows[4 * i + u, sl]
            return inner
        lax.fori_loop(0, CH // 4, mrow, 0)

        cps = [
            pltpu.async_copy(x_rows.at[pl.ds(0, 128)], acc.at[d0], sem, add=True),
            pltpu.async_copy(x_rows.at[pl.ds(128, 128)], acc.at[d1], sem, add=True),
            pltpu.async_copy(x_rows.at[pl.ds(256, 64)], acc.at[d2], sem, add=True),
        ]
        for cp in cps:
            cp.wait()
        return carry
    lax.fori_loop(0, nk, chunk, 0)

    plsc.subcore_barrier()
    pltpu.sync_copy(acc.at[pl.ds(sid * OUT_PER_TILE, OUT_PER_TILE)],
                    out_hbm.at[pl.ds(sid * OUT_PER_TILE, OUT_PER_TILE)])
    @pl.when(sid == 0)
    def _tail():
        pltpu.sync_copy(acc.at[pl.ds(NS * OUT_PER_TILE, N - NS * OUT_PER_TILE)],
                        out_hbm.at[pl.ds(NS * OUT_PER_TILE, N - NS * OUT_PER_TILE)])


def _make_sc(phase):
    base_e = PHASE_BASE_E[phase]
    totch = PHASE_EDGES[phase] // CH
    chained = phase == 1

    scratch = [
        pltpu.VMEM((CH, HDIM), jnp.float32),
        pltpu.VMEM((CH, HDIM), jnp.float32),
        pltpu.VMEM((CH,), jnp.int32),
        pltpu.VMEM((128,), jnp.int32),
        pltpu.VMEM((128,), jnp.int32),
        pltpu.VMEM((64,), jnp.int32),
        pltpu.VMEM_SHARED((N, HDIM), jnp.float32),
        pltpu.SemaphoreType.DMA,
    ]
    kwargs = dict(
        out_type=[
            jax.ShapeDtypeStruct((N, HDIM), jnp.float32),
            jax.ShapeDtypeStruct((N, HDIM), jnp.float32),
        ],
        mesh=plsc.VectorSubcoreMesh(core_axis_name="c", subcore_axis_name="s"),
        scratch_types=scratch,
        compiler_params=pltpu.CompilerParams(use_tc_tiling_on_sc=False),
    )

    if not chained:
        @functools.partial(pl.kernel, **kwargs)
        def sc(x0_hbm, x1_hbm, src_hbm, dst_hbm, h0_hbm, h1_hbm,
               out0_hbm, out1_hbm,
               x_rows, h_rows, src_v, d0, d1, d2, acc, sem):
            cid = lax.axis_index("c")
            sid = lax.axis_index("s")

            @pl.when(cid == 0)
            def _half0():
                _sc_half(x0_hbm, src_hbm, dst_hbm, h0_hbm, out0_hbm,
                         x_rows, h_rows, src_v, d0, d1, d2, acc, sem, sid,
                         base_e, totch)

            @pl.when(cid == 1)
            def _half1():
                _sc_half(x1_hbm, src_hbm, dst_hbm, h1_hbm, out1_hbm,
                         x_rows, h_rows, src_v, d0, d1, d2, acc, sem, sid,
                         base_e, totch)
        return sc

    @functools.partial(pl.kernel, **kwargs)
    def sc_chained(x0_hbm, x1_hbm, src_hbm, dst_hbm, h0_hbm, h1_hbm,
                   p0_hbm, p1_hbm, out0_hbm, out1_hbm,
                   x_rows, h_rows, src_v, d0, d1, d2, acc, sem):
        cid = lax.axis_index("c")
        sid = lax.axis_index("s")

        @pl.when(cid == 0)
        def _half0():
            _sc_half(x0_hbm, src_hbm, dst_hbm, h0_hbm, out0_hbm,
                     x_rows, h_rows, src_v, d0, d1, d2, acc, sem, sid,
                     base_e, totch, prev_hbm=p0_hbm)

        @pl.when(cid == 1)
        def _half1():
            _sc_half(x1_hbm, src_hbm, dst_hbm, h1_hbm, out1_hbm,
                     x_rows, h_rows, src_v, d0, d1, d2, acc, sem, sid,
                     base_e, totch, prev_hbm=p1_hbm)
    return sc_chained


_mlp_a = _make_edge_mlp(PHASE_BASE_BLK[0])
_mlp_b = _make_edge_mlp(PHASE_BASE_BLK[1])
_sc_a = _make_sc(0)
_sc_b = _make_sc(1)


def kernel(x, edge_index, rbf, W1, b1, W2, b2):
    rbf_t = rbf.T
    b1c = b1.reshape(DIM, 1)
    b2r = b2.reshape(1, DIM)
    src = edge_index[0].astype(jnp.int32)
    dst = edge_index[1].astype(jnp.int32)
    x0 = x[:, :HDIM]
    x1 = x[:, HDIM:]
    ha0, ha1 = _mlp_a(rbf_t, W1, b1c, W2, b2r)
    hb0, hb1 = _mlp_b(rbf_t, W1, b1c, W2, b2r)
    oa0, oa1 = _sc_a(x0, x1, src, dst, ha0, ha1)
    ob0, ob1 = _sc_b(x0, x1, src, dst, hb0, hb1, oa0, oa1)
    return jnp.concatenate([ob0, ob1], axis=1)


# speedup vs baseline: 4.9842x; 1.1369x over previous
import functools

import jax
import jax.numpy as jnp
from jax import lax
from jax.experimental import pallas as pl
from jax.experimental.pallas import tpu as pltpu
from jax.experimental.pallas import tpu_sc as plsc

N = 50000
E = 800000
DIM = 64

NC = 2
NS = 16
LANES = 16

HDIM = DIM // NC

BEB = 8192
GBLK = 15
FGG = BEB * GBLK
LASTBLK = (E - 1) // BEB

PHASE_BASE_BLK = (0, 50)
PHASE_BASE_E = (0, 50 * BEB)
PHASE_EDGES = (50 * BEB, E - 50 * BEB)

CH = 160
CHA = 128
CHB = CH - CHA
CPG = FGG // CH

OUT_PER_TILE = 3120
ZROWS = CH
ZFULL = (N // NS) // ZROWS
ZTAIL = N // NS - ZFULL * ZROWS


def _mlp_block(r0_ref, r1_ref, r2_ref, r3_ref, w1_ref, b1_ref, w2_ref, b2_ref,
               h0_ref, h1_ref):
    halves0, halves1 = [], []
    for rt_ref in (r0_ref, r1_ref, r2_ref, r3_ref):
        r_t = rt_ref[...]
        a = jnp.dot(w1_ref[...], r_t, preferred_element_type=jnp.float32) + b1_ref[...]
        a = jnp.where(0.5 * a > 14.0, a, 2.0 * jnp.log1p(jnp.exp(0.5 * a)))
        h = lax.dot_general(a, w2_ref[...], (((0,), (1,)), ((), ())),
                            preferred_element_type=jnp.float32) + b2_ref[...]
        halves0.append(h[:, :HDIM])
        halves1.append(h[:, HDIM:])
    h0_ref[...] = jnp.concatenate(halves0, axis=1)
    h1_ref[...] = jnp.concatenate(halves1, axis=1)


def _make_edge_mlp(base_blk):
    def _rspec(g):
        return pl.BlockSpec(
            (DIM, BEB),
            lambda i, g=g: (0, jnp.minimum(base_blk + g * GBLK + i, LASTBLK)))

    def run(rbf_t, w1, b1c, w2, b2r):
        return pl.pallas_call(
            _mlp_block,
            grid=(GBLK,),
            in_specs=[
                _rspec(0), _rspec(1), _rspec(2), _rspec(3),
                pl.BlockSpec((DIM, DIM), lambda i: (0, 0)),
                pl.BlockSpec((DIM, 1), lambda i: (0, 0)),
                pl.BlockSpec((DIM, DIM), lambda i: (0, 0)),
                pl.BlockSpec((1, DIM), lambda i: (0, 0)),
            ],
            out_specs=[
                pl.BlockSpec((BEB, 128), lambda i: (i, 0)),
                pl.BlockSpec((BEB, 128), lambda i: (i, 0)),
            ],
            out_shape=[
                jax.ShapeDtypeStruct((FGG, 128), jnp.float32),
                jax.ShapeDtypeStruct((FGG, 128), jnp.float32),
            ],
        )(rbf_t, rbf_t, rbf_t, rbf_t, w1, b1c, w2, b2r)
    return run


def _sc_half(x_hbm, src_hbm, dst_hbm, h_hbm, out_hbm,
             bufs, acc, sid, base_e, totch, prev_hbm=None):
    (buf0, buf1), semG = bufs
    zbase = sid * (N // NS)
    if prev_hbm is None:
        xr0 = buf0[0]

        def zrow(r, carry):
            for q in range(HDIM // LANES):
                xr0[r, pl.ds(q * LANES, LANES)] = jnp.zeros((LANES,), jnp.float32)
            return carry
        lax.fori_loop(0, ZROWS, zrow, 0)

        def zcopy(k, carry):
            pltpu.sync_copy(xr0.at[pl.ds(0, ZROWS)],
                            acc.at[pl.ds(zbase + k * ZROWS, ZROWS)])
            return carry
        lax.fori_loop(0, ZFULL, zcopy, 0)
        pltpu.sync_copy(xr0.at[pl.ds(0, ZTAIL)],
                        acc.at[pl.ds(zbase + ZFULL * ZROWS, ZTAIL)])
    else:
        pltpu.sync_copy(prev_hbm.at[pl.ds(zbase, N // NS)],
                        acc.at[pl.ds(zbase, N // NS)])
    plsc.subcore_barrier()

    nk = totch // NS + jnp.where(sid < totch % NS, 1, 0)

    def issue_wave1(kk, buf):
        xr, hr, sv, dA, dB, semL, semS = buf
        k = kk * NS + sid
        gl = k // CPG
        j = k - gl * CPG
        base = base_e + gl * FGG + j * CH
        base_r = j * CH
        pltpu.async_copy(src_hbm.at[pl.ds(base, CH)], sv, semL)
        pltpu.async_copy(dst_hbm.at[pl.ds(base, CHA)], dA, semL)
        pltpu.async_copy(dst_hbm.at[pl.ds(base + CHA, CHB)], dB, semL)
        pltpu.async_copy(
            h_hbm.at[pl.ds(base_r, CH), pl.ds(gl * HDIM, HDIM)], hr, semL)

    def drain_wave1(buf):
        xr, hr, sv, dA, dB, semL, semS = buf
        pltpu.make_async_copy(src_hbm.at[pl.ds(0, CH)], sv, semL).wait()
        pltpu.make_async_copy(dst_hbm.at[pl.ds(0, CHA)], dA, semL).wait()
        pltpu.make_async_copy(dst_hbm.at[pl.ds(0, CHB)], dB, semL).wait()
        pltpu.make_async_copy(
            h_hbm.at[pl.ds(0, CH), pl.ds(0, HDIM)], hr, semL).wait()

    def drain_scatter(buf):
        xr, hr, sv, dA, dB, semL, semS = buf
        pltpu.make_async_copy(
            h_hbm.at[pl.ds(0, CHA), pl.ds(0, HDIM)], xr.at[pl.ds(0, CHA)],
            semS).wait()
        pltpu.make_async_copy(
            h_hbm.at[pl.ds(0, CHB), pl.ds(0, HDIM)], xr.at[pl.ds(CHA, CHB)],
            semS).wait()

    @pl.when(nk > 0)
    def _prime():
        issue_wave1(0, buf0)

    def body(ii, carry):
        for b, buf, obuf in ((0, buf0, buf1), (1, buf1, buf0)):
            kk = 2 * ii + b
            xr, hr, sv, dA, dB, semL, semS = buf

            @pl.when(kk < nk)
            def _step():
                drain_wave1(buf)
                g1 = pltpu.async_copy(x_hbm.at[sv.at[pl.ds(0, CHA)]],
                                      xr.at[pl.ds(0, CHA)], semG)
                g2 = pltpu.async_copy(x_hbm.at[sv.at[pl.ds(CHA, CHB)]],
                                      xr.at[pl.ds(CHA, CHB)], semG)
                @pl.when(kk >= 1)
                def _ds():
                    drain_scatter(obuf)

                @pl.when(kk + 1 < nk)
                def _next():
                    issue_wave1(kk + 1, obuf)
                g1.wait()
                g2.wait()

                def mrow(i, inner):
                    for u in range(4):
                        for q in range(HDIM // LANES):
                            sl = pl.ds(q * LANES, LANES)
                            xr[4 * i + u, sl] = xr[4 * i + u, sl] * hr[4 * i + u, sl]
                    return inner
                lax.fori_loop(0, CH // 4, mrow, 0)

                pltpu.async_copy(xr.at[pl.ds(0, CHA)], acc.at[dA], semS, add=True)
                pltpu.async_copy(xr.at[pl.ds(CHA, CHB)], acc.at[dB], semS, add=True)
        return carry
    lax.fori_loop(0, (nk + 1) // 2, body, 0)

    @pl.when(jnp.logical_and(nk >= 1, (nk - 1) % 2 == 0))
    def _tail0():
        drain_scatter(buf0)

    @pl.when(jnp.logical_and(nk >= 2, (nk - 1) % 2 == 1))
    def _tail1():
        drain_scatter(buf1)

    plsc.subcore_barrier()
    pltpu.sync_copy(acc.at[pl.ds(sid * OUT_PER_TILE, OUT_PER_TILE)],
                    out_hbm.at[pl.ds(sid * OUT_PER_TILE, OUT_PER_TILE)])
    @pl.when(sid == 0)
    def _tail():
        pltpu.sync_copy(acc.at[pl.ds(NS * OUT_PER_TILE, N - NS * OUT_PER_TILE)],
                        out_hbm.at[pl.ds(NS * OUT_PER_TILE, N - NS * OUT_PER_TILE)])


def _make_sc(phase):
    base_e = PHASE_BASE_E[phase]
    totch = PHASE_EDGES[phase] // CH
    chained = phase == 1

    parity = [
        pltpu.VMEM((CH, HDIM), jnp.float32),
        pltpu.VMEM((CH, HDIM), jnp.float32),
        pltpu.VMEM((CH,), jnp.int32),
        pltpu.VMEM((CHA,), jnp.int32),
        pltpu.VMEM((CHB,), jnp.int32),
        pltpu.SemaphoreType.DMA,
        pltpu.SemaphoreType.DMA,
    ]
    scratch = parity + parity + [
        pltpu.SemaphoreType.DMA,
        pltpu.VMEM_SHARED((N, HDIM), jnp.float32),
    ]
    kwargs = dict(
        out_type=[
            jax.ShapeDtypeStruct((N, HDIM), jnp.float32),
            jax.ShapeDtypeStruct((N, HDIM), jnp.float32),
        ],
        mesh=plsc.VectorSubcoreMesh(core_axis_name="c", subcore_axis_name="s"),
        scratch_types=scratch,
        compiler_params=pltpu.CompilerParams(use_tc_tiling_on_sc=False),
    )

    if not chained:
        @functools.partial(pl.kernel, **kwargs)
        def sc(x0_hbm, x1_hbm, src_hbm, dst_hbm, h0_hbm, h1_hbm,
               out0_hbm, out1_hbm,
               xr0, hr0, sv0, dA0, dB0, semL0, semS0,
               xr1, hr1, sv1, dA1, dB1, semL1, semS1, semG, acc):
            cid = lax.axis_index("c")
            sid = lax.axis_index("s")
            bufs = (((xr0, hr0, sv0, dA0, dB0, semL0, semS0),
                     (xr1, hr1, sv1, dA1, dB1, semL1, semS1)), semG)

            @pl.when(cid == 0)
            def _half0():
                _sc_half(x0_hbm, src_hbm, dst_hbm, h0_hbm, out0_hbm,
                         bufs, acc, sid, base_e, totch)

            @pl.when(cid == 1)
            def _half1():
                _sc_half(x1_hbm, src_hbm, dst_hbm, h1_hbm, out1_hbm,
                         bufs, acc, sid, base_e, totch)
        return sc

    @functools.partial(pl.kernel, **kwargs)
    def sc_chained(x0_hbm, x1_hbm, src_hbm, dst_hbm, h0_hbm, h1_hbm,
                   p0_hbm, p1_hbm, out0_hbm, out1_hbm,
                   xr0, hr0, sv0, dA0, dB0, semL0, semS0,
                   xr1, hr1, sv1, dA1, dB1, semL1, semS1, semG, acc):
        cid = lax.axis_index("c")
        sid = lax.axis_index("s")
        bufs = (((xr0, hr0, sv0, dA0, dB0, semL0, semS0),
                 (xr1, hr1, sv1, dA1, dB1, semL1, semS1)), semG)

        @pl.when(cid == 0)
        def _half0():
            _sc_half(x0_hbm, src_hbm, dst_hbm, h0_hbm, out0_hbm,
                     bufs, acc, sid, base_e, totch, prev_hbm=p0_hbm)

        @pl.when(cid == 1)
        def _half1():
            _sc_half(x1_hbm, src_hbm, dst_hbm, h1_hbm, out1_hbm,
                     bufs, acc, sid, base_e, totch, prev_hbm=p1_hbm)
    return sc_chained


_mlp_a = _make_edge_mlp(PHASE_BASE_BLK[0])
_mlp_b = _make_edge_mlp(PHASE_BASE_BLK[1])
_sc_a = _make_sc(0)
_sc_b = _make_sc(1)


def kernel(x, edge_index, rbf, W1, b1, W2, b2):
    rbf_t = rbf.T
    b1c = b1.reshape(DIM, 1)
    b2r = b2.reshape(1, DIM)
    src = edge_index[0].astype(jnp.int32)
    dst = edge_index[1].astype(jnp.int32)
    x0 = x[:, :HDIM]
    x1 = x[:, HDIM:]
    ha0, ha1 = _mlp_a(rbf_t, W1, b1c, W2, b2r)
    hb0, hb1 = _mlp_b(rbf_t, W1, b1c, W2, b2r)
    oa0, oa1 = _sc_a(x0, x1, src, dst, ha0, ha1)
    ob0, ob1 = _sc_b(x0, x1, src, dst, hb0, hb1, oa0, oa1)
    return jnp.concatenate([ob0, ob1], axis=1)


# speedup vs baseline: 5.1775x; 1.0388x over previous
import functools

import jax
import jax.numpy as jnp
from jax import lax
from jax.experimental import pallas as pl
from jax.experimental.pallas import tpu as pltpu
from jax.experimental.pallas import tpu_sc as plsc

N = 50000
E = 800000
DIM = 64

NC = 2
NS = 16
LANES = 16

HDIM = DIM // NC

BEB = 8192
GBLK = 15
FGG = BEB * GBLK
LASTBLK = (E - 1) // BEB

PHASE_BASE_BLK = (0, 50)
PHASE_BASE_E = (0, 50 * BEB)
PHASE_EDGES = (50 * BEB, E - 50 * BEB)

CH = 160
CHA = 128
CHB = CH - CHA
CPG = FGG // CH

OUT_PER_TILE = 3120
ZROWS = CH
ZFULL = (N // NS) // ZROWS
ZTAIL = N // NS - ZFULL * ZROWS


def _mlp_block(r0_ref, r1_ref, r2_ref, r3_ref, w1_ref, b1_ref, w2_ref, b2_ref,
               h0_ref, h1_ref):
    halves0, halves1 = [], []
    for rt_ref in (r0_ref, r1_ref, r2_ref, r3_ref):
        r_t = rt_ref[...]
        a = jnp.dot(w1_ref[...], r_t, preferred_element_type=jnp.float32) + b1_ref[...]
        a = jnp.where(0.5 * a > 14.0, a, 2.0 * jnp.log1p(jnp.exp(0.5 * a)))
        h = lax.dot_general(a, w2_ref[...], (((0,), (1,)), ((), ())),
                            preferred_element_type=jnp.float32) + b2_ref[...]
        halves0.append(h[:, :HDIM])
        halves1.append(h[:, HDIM:])
    h0_ref[...] = jnp.concatenate(halves0, axis=1)
    h1_ref[...] = jnp.concatenate(halves1, axis=1)


def _make_edge_mlp(base_blk):
    def _rspec(g):
        return pl.BlockSpec(
            (DIM, BEB),
            lambda i, g=g: (0, jnp.minimum(base_blk + g * GBLK + i, LASTBLK)))

    def run(rbf_t, w1, b1c, w2, b2r):
        return pl.pallas_call(
            _mlp_block,
            grid=(GBLK,),
            in_specs=[
                _rspec(0), _rspec(1), _rspec(2), _rspec(3),
                pl.BlockSpec((DIM, DIM), lambda i: (0, 0)),
                pl.BlockSpec((DIM, 1), lambda i: (0, 0)),
                pl.BlockSpec((DIM, DIM), lambda i: (0, 0)),
                pl.BlockSpec((1, DIM), lambda i: (0, 0)),
            ],
            out_specs=[
                pl.BlockSpec((BEB, 128), lambda i: (i, 0)),
                pl.BlockSpec((BEB, 128), lambda i: (i, 0)),
            ],
            out_shape=[
                jax.ShapeDtypeStruct((FGG, 128), jnp.float32),
                jax.ShapeDtypeStruct((FGG, 128), jnp.float32),
            ],
        )(rbf_t, rbf_t, rbf_t, rbf_t, w1, b1c, w2, b2r)
    return run


def _sc_half(x_hbm, src_hbm, dst_hbm, h_hbm, out_hbm,
             bufs, acc, sid, base_e, totch, prev_hbm=None, full_out=False):
    (buf0, buf1), semG = bufs
    zbase = sid * (N // NS)
    if prev_hbm is None:
        xr0 = buf0[0]

        def zrow(r, carry):
            for q in range(HDIM // LANES):
                xr0[r, pl.ds(q * LANES, LANES)] = jnp.zeros((LANES,), jnp.float32)
            return carry
        lax.fori_loop(0, ZROWS, zrow, 0)

        def zcopy(k, carry):
            pltpu.sync_copy(xr0.at[pl.ds(0, ZROWS)],
                            acc.at[pl.ds(zbase + k * ZROWS, ZROWS)])
            return carry
        lax.fori_loop(0, ZFULL, zcopy, 0)
        pltpu.sync_copy(xr0.at[pl.ds(0, ZTAIL)],
                        acc.at[pl.ds(zbase + ZFULL * ZROWS, ZTAIL)])
    else:
        pltpu.sync_copy(prev_hbm.at[pl.ds(zbase, N // NS)],
                        acc.at[pl.ds(zbase, N // NS)])
    plsc.subcore_barrier()

    nk = totch // NS + jnp.where(sid < totch % NS, 1, 0)

    def issue_wave1(kk, buf):
        xr, hr, sv, dA, dB, semL, semS = buf
        k = kk * NS + sid
        gl = k // CPG
        j = k - gl * CPG
        base = base_e + gl * FGG + j * CH
        base_r = j * CH
        pltpu.async_copy(src_hbm.at[pl.ds(base, CH)], sv, semL)
        pltpu.async_copy(dst_hbm.at[pl.ds(base, CHA)], dA, semL)
        pltpu.async_copy(dst_hbm.at[pl.ds(base + CHA, CHB)], dB, semL)
        pltpu.async_copy(
            h_hbm.at[pl.ds(base_r, CH), pl.ds(gl * HDIM, HDIM)], hr, semL)

    def drain_wave1(buf):
        xr, hr, sv, dA, dB, semL, semS = buf
        pltpu.make_async_copy(src_hbm.at[pl.ds(0, CH)], sv, semL).wait()
        pltpu.make_async_copy(dst_hbm.at[pl.ds(0, CHA)], dA, semL).wait()
        pltpu.make_async_copy(dst_hbm.at[pl.ds(0, CHB)], dB, semL).wait()
        pltpu.make_async_copy(
            h_hbm.at[pl.ds(0, CH), pl.ds(0, HDIM)], hr, semL).wait()

    def drain_scatter(buf):
        xr, hr, sv, dA, dB, semL, semS = buf
        pltpu.make_async_copy(
            h_hbm.at[pl.ds(0, CHA), pl.ds(0, HDIM)], xr.at[pl.ds(0, CHA)],
            semS).wait()
        pltpu.make_async_copy(
            h_hbm.at[pl.ds(0, CHB), pl.ds(0, HDIM)], xr.at[pl.ds(CHA, CHB)],
            semS).wait()

    @pl.when(nk > 0)
    def _prime():
        issue_wave1(0, buf0)

    def body(ii, carry):
        for b, buf, obuf in ((0, buf0, buf1), (1, buf1, buf0)):
            kk = 2 * ii + b
            xr, hr, sv, dA, dB, semL, semS = buf

            @pl.when(kk < nk)
            def _step():
                drain_wave1(buf)
                g1 = pltpu.async_copy(x_hbm.at[sv.at[pl.ds(0, CHA)]],
                                      xr.at[pl.ds(0, CHA)], semG)
                g2 = pltpu.async_copy(x_hbm.at[sv.at[pl.ds(CHA, CHB)]],
                                      xr.at[pl.ds(CHA, CHB)], semG)
                @pl.when(kk >= 1)
                def _ds():
                    drain_scatter(obuf)

                @pl.when(kk + 1 < nk)
                def _next():
                    issue_wave1(kk + 1, obuf)
                g1.wait()
                g2.wait()

                def mrow(i, inner):
                    for u in range(4):
                        for q in range(HDIM // LANES):
                            sl = pl.ds(q * LANES, LANES)
                            xr[4 * i + u, sl] = xr[4 * i + u, sl] * hr[4 * i + u, sl]
                    return inner
                lax.fori_loop(0, CH // 4, mrow, 0)

                pltpu.async_copy(xr.at[pl.ds(0, CHA)], acc.at[dA], semS, add=True)
                pltpu.async_copy(xr.at[pl.ds(CHA, CHB)], acc.at[dB], semS, add=True)
        return carry
    lax.fori_loop(0, (nk + 1) // 2, body, 0)

    @pl.when(jnp.logical_and(nk >= 1, (nk - 1) % 2 == 0))
    def _tail0():
        drain_scatter(buf0)

    @pl.when(jnp.logical_and(nk >= 2, (nk - 1) % 2 == 1))
    def _tail1():
        drain_scatter(buf1)

    plsc.subcore_barrier()
    if full_out:
        cid = lax.axis_index("c")
        col = cid * HDIM
        pltpu.sync_copy(
            acc.at[pl.ds(sid * OUT_PER_TILE, OUT_PER_TILE)],
            out_hbm.at[pl.ds(sid * OUT_PER_TILE, OUT_PER_TILE), pl.ds(col, HDIM)])
        @pl.when(sid == 0)
        def _tail():
            pltpu.sync_copy(
                acc.at[pl.ds(NS * OUT_PER_TILE, N - NS * OUT_PER_TILE)],
                out_hbm.at[pl.ds(NS * OUT_PER_TILE, N - NS * OUT_PER_TILE),
                           pl.ds(col, HDIM)])
    else:
        pltpu.sync_copy(acc.at[pl.ds(sid * OUT_PER_TILE, OUT_PER_TILE)],
                        out_hbm.at[pl.ds(sid * OUT_PER_TILE, OUT_PER_TILE)])
        @pl.when(sid == 0)
        def _tail():
            pltpu.sync_copy(acc.at[pl.ds(NS * OUT_PER_TILE, N - NS * OUT_PER_TILE)],
                            out_hbm.at[pl.ds(NS * OUT_PER_TILE, N - NS * OUT_PER_TILE)])


def _make_sc(phase):
    base_e = PHASE_BASE_E[phase]
    totch = PHASE_EDGES[phase] // CH
    chained = phase == 1

    parity = [
        pltpu.VMEM((CH, HDIM), jnp.float32),
        pltpu.VMEM((CH, HDIM), jnp.float32),
        pltpu.VMEM((CH,), jnp.int32),
        pltpu.VMEM((CHA,), jnp.int32),
        pltpu.VMEM((CHB,), jnp.int32),
        pltpu.SemaphoreType.DMA,
        pltpu.SemaphoreType.DMA,
    ]
    scratch = parity + parity + [
        pltpu.SemaphoreType.DMA,
        pltpu.VMEM_SHARED((N, HDIM), jnp.float32),
    ]
    kwargs = dict(
        out_type=[
            jax.ShapeDtypeStruct((N, HDIM), jnp.float32),
            jax.ShapeDtypeStruct((N, HDIM), jnp.float32),
        ],
        mesh=plsc.VectorSubcoreMesh(core_axis_name="c", subcore_axis_name="s"),
        scratch_types=scratch,
        compiler_params=pltpu.CompilerParams(use_tc_tiling_on_sc=False),
    )

    if not chained:
        @functools.partial(pl.kernel, **kwargs)
        def sc(x0_hbm, x1_hbm, src_hbm, dst_hbm, h0_hbm, h1_hbm,
               out0_hbm, out1_hbm,
               xr0, hr0, sv0, dA0, dB0, semL0, semS0,
               xr1, hr1, sv1, dA1, dB1, semL1, semS1, semG, acc):
            cid = lax.axis_index("c")
            sid = lax.axis_index("s")
            bufs = (((xr0, hr0, sv0, dA0, dB0, semL0, semS0),
                     (xr1, hr1, sv1, dA1, dB1, semL1, semS1)), semG)

            @pl.when(cid == 0)
            def _half0():
                _sc_half(x0_hbm, src_hbm, dst_hbm, h0_hbm, out0_hbm,
                         bufs, acc, sid, base_e, totch)

            @pl.when(cid == 1)
            def _half1():
                _sc_half(x1_hbm, src_hbm, dst_hbm, h1_hbm, out1_hbm,
                         bufs, acc, sid, base_e, totch)
        return sc

    kwargs["out_type"] = jax.ShapeDtypeStruct((N, DIM), jnp.float32)

    @functools.partial(pl.kernel, **kwargs)
    def sc_chained(x0_hbm, x1_hbm, src_hbm, dst_hbm, h0_hbm, h1_hbm,
                   p0_hbm, p1_hbm, out_hbm,
                   xr0, hr0, sv0, dA0, dB0, semL0, semS0,
                   xr1, hr1, sv1, dA1, dB1, semL1, semS1, semG, acc):
        cid = lax.axis_index("c")
        sid = lax.axis_index("s")
        bufs = (((xr0, hr0, sv0, dA0, dB0, semL0, semS0),
                 (xr1, hr1, sv1, dA1, dB1, semL1, semS1)), semG)

        @pl.when(cid == 0)
        def _half0():
            _sc_half(x0_hbm, src_hbm, dst_hbm, h0_hbm, out_hbm,
                     bufs, acc, sid, base_e, totch, prev_hbm=p0_hbm,
                     full_out=True)

        @pl.when(cid == 1)
        def _half1():
            _sc_half(x1_hbm, src_hbm, dst_hbm, h1_hbm, out_hbm,
                     bufs, acc, sid, base_e, totch, prev_hbm=p1_hbm,
                     full_out=True)
    return sc_chained


_mlp_a = _make_edge_mlp(PHASE_BASE_BLK[0])
_mlp_b = _make_edge_mlp(PHASE_BASE_BLK[1])
_sc_a = _make_sc(0)
_sc_b = _make_sc(1)


def kernel(x, edge_index, rbf, W1, b1, W2, b2):
    rbf_t = rbf.T
    b1c = b1.reshape(DIM, 1)
    b2r = b2.reshape(1, DIM)
    src = edge_index[0].astype(jnp.int32)
    dst = edge_index[1].astype(jnp.int32)
    x0 = x[:, :HDIM]
    x1 = x[:, HDIM:]
    ha0, ha1 = _mlp_a(rbf_t, W1, b1c, W2, b2r)
    hb0, hb1 = _mlp_b(rbf_t, W1, b1c, W2, b2r)
    oa0, oa1 = _sc_a(x0, x1, src, dst, ha0, ha1)
    return _sc_b(x0, x1, src, dst, hb0, hb1, oa0, oa1)
